# Initial kernel scaffold; baseline (speedup 1.0000x reference)
import functools

import jax
import jax.numpy as jnp
from jax import lax
from jax.experimental import pallas as pl
from jax.experimental.pallas import tpu as pltpu
from jax.experimental.pallas import tpu_sc as plsc

N = 10000
E = 320000
D = 128
DE = 16
H = 128

NC = 2
NS = 16
NW = NC * NS
L = 16

NPAD = 10240
RPT = NPAD // NS
C = 128
NCHUNK = E // C
CPT = -(-NCHUNK // NW)

BR = 256
GRID = NPAD // BR

EB = E * DE // 128
BR4 = 800
GRID4 = EB // BR4


def _sc_degree(dst, ones):
    mesh = plsc.VectorSubcoreMesh(core_axis_name="c", subcore_axis_name="s")

    @functools.partial(
        pl.kernel,
        out_type=jax.ShapeDtypeStruct((NC, NPAD), jnp.float32),
        mesh=mesh,
        scratch_types=[
            pltpu.VMEM_SHARED((NPAD,), jnp.float32),
            pltpu.VMEM((C,), jnp.int32),
            pltpu.VMEM((C,), jnp.float32),
        ],
    )
    def k(dst_hbm, ones_hbm, out_hbm, dsh, didx, ones_v):
        c = lax.axis_index("c")
        s = lax.axis_index("s")
        wid = s * NC + c
        rbase = s * RPT
        pltpu.sync_copy(ones_hbm.at[pl.ds(rbase, RPT)], dsh.at[pl.ds(rbase, RPT)])
        pltpu.sync_copy(ones_hbm.at[pl.ds(0, C)], ones_v)
        plsc.subcore_barrier()

        def body(j, carry):
            cid = j * NW + wid

            @pl.when(cid < NCHUNK)
            def _():
                pltpu.sync_copy(dst_hbm.at[pl.ds(cid * C, C)], didx)
                pltpu.sync_copy(ones_v, dsh.at[didx], add=True)

            return carry

        lax.fori_loop(0, CPT, body, 0)
        plsc.subcore_barrier()
        pltpu.sync_copy(dsh.at[pl.ds(rbase, RPT)],
                        out_hbm.at[c].at[pl.ds(rbase, RPT)])

    return k(dst, ones)


def _sc_spmm(hs, src, dst):
    mesh = plsc.VectorSubcoreMesh(core_axis_name="c", subcore_axis_name="s")

    @functools.partial(
        pl.kernel,
        out_type=jax.ShapeDtypeStruct((NC, NPAD, D), jnp.float32),
        mesh=mesh,
        scratch_types=[
            pltpu.VMEM_SHARED((NPAD, D), jnp.float32),
            pltpu.VMEM((C,), jnp.int32),
            pltpu.VMEM((C,), jnp.int32),
            pltpu.VMEM((C, D), jnp.float32),
            pltpu.SemaphoreType.DMA,
        ],
    )
    def k(hs_hbm, src_hbm, dst_hbm, out_hbm, ysh, sidx, didx, rows, sem):
        c = lax.axis_index("c")
        s = lax.axis_index("s")
        wid = s * NC + c
        rbase = s * RPT
        pltpu.sync_copy(hs_hbm.at[pl.ds(rbase, RPT)], ysh.at[pl.ds(rbase, RPT)])
        plsc.subcore_barrier()

        def body(j, carry):
            cid = j * NW + wid

            @pl.when(cid < NCHUNK)
            def _():
                start = cid * C
                pltpu.sync_copy(src_hbm.at[pl.ds(start, C)], sidx)
                pltpu.sync_copy(dst_hbm.at[pl.ds(start, C)], didx)
                pltpu.async_copy(hs_hbm.at[sidx], rows, sem).wait()
                pltpu.sync_copy(rows, ysh.at[didx], add=True)

            return carry

        lax.fori_loop(0, CPT, body, 0)
        plsc.subcore_barrier()
        pltpu.sync_copy(ysh.at[pl.ds(rbase, RPT)],
                        out_hbm.at[c].at[pl.ds(rbase, RPT)])

    return k(hs, src, dst)


def _sc_edge_logits(a, b, src, dst, et):
    mesh = plsc.VectorSubcoreMesh(core_axis_name="c", subcore_axis_name="s")

    @functools.partial(
        pl.kernel,
        out_type=jax.ShapeDtypeStruct((E,), jnp.float32),
        mesh=mesh,
        scratch_types=[
            pltpu.VMEM((NPAD,), jnp.float32),
            pltpu.VMEM((NPAD,), jnp.float32),
            pltpu.VMEM((C,), jnp.int32),
            pltpu.VMEM((C,), jnp.int32),
            pltpu.VMEM((C,), jnp.float32),
            pltpu.VMEM((C,), jnp.float32),
        ],
    )
    def k(a_hbm, b_hbm, src_hbm, dst_hbm, et_hbm, out_hbm,
          av, bv, sidx, didx, etv, ov):
        c = lax.axis_index("c")
        s = lax.axis_index("s")
        wid = s * NC + c
        pltpu.sync_copy(a_hbm, av)
        pltpu.sync_copy(b_hbm, bv)

        def body(j, carry):
            cid = j * NW + wid

            @pl.when(cid < NCHUNK)
            def _():
                start = cid * C
                pltpu.sync_copy(src_hbm.at[pl.ds(start, C)], sidx)
                pltpu.sync_copy(dst_hbm.at[pl.ds(start, C)], didx)
                pltpu.sync_copy(et_hbm.at[pl.ds(start, C)], etv)
                for kk in range(C // L):
                    i16s = sidx[pl.ds(kk * L, L)]
                    i16d = didx[pl.ds(kk * L, L)]
                    ga = plsc.load_gather(av, [i16s])
                    gb = plsc.load_gather(bv, [i16d])
                    ov[pl.ds(kk * L, L)] = ga + gb + etv[pl.ds(kk * L, L)]
                pltpu.sync_copy(ov, out_hbm.at[pl.ds(start, C)])

            return carry

        lax.fori_loop(0, CPT, body, 0)

    return k(a, b, src, dst, et)


def _dinv_block(dega, degb):
    deg = dega + degb - 1.0
    return lax.rsqrt(jnp.maximum(deg, 1.0))


def _k_mm_scale_body(x_ref, w_ref, da_ref, db_ref, o_ref):
    dinv = _dinv_block(da_ref[...], db_ref[...])
    h = jnp.dot(x_ref[...], w_ref[...], preferred_element_type=jnp.float32)
    o_ref[...] = h * dinv


def _k_mm_scale(x, w, dega, degb):
    return pl.pallas_call(
        _k_mm_scale_body,
        grid=(GRID,),
        in_specs=[
            pl.BlockSpec((BR, D), lambda i: (i, 0)),
            pl.BlockSpec((D, H), lambda i: (0, 0)),
            pl.BlockSpec((BR, 1), lambda i: (i, 0)),
            pl.BlockSpec((BR, 1), lambda i: (i, 0)),
        ],
        out_specs=pl.BlockSpec((BR, H), lambda i: (i, 0)),
        out_shape=jax.ShapeDtypeStruct((NPAD, H), jnp.float32),
    )(x, w, dega, degb)


def _k_combine_stats_body(pa_ref, pb_ref, hs_ref, da_ref, db_ref, b_ref,
                          y_ref, st_ref):
    i = pl.program_id(0)
    dinv = _dinv_block(da_ref[...], db_ref[...])
    y = dinv * (pa_ref[...] + pb_ref[...] - hs_ref[...]) + b_ref[...]
    y_ref[...] = y

    @pl.when(i == 0)
    def _():
        st_ref[...] = jnp.zeros_like(st_ref)

    gid = i * BR + lax.broadcasted_iota(jnp.int32, (BR, 1), 0)
    ym = jnp.where(gid < N, y, 0.0)
    st_ref[0:1, :] += jnp.sum(ym, axis=0, keepdims=True)
    st_ref[1:2, :] += jnp.sum(ym * ym, axis=0, keepdims=True)


def _k_combine_stats(pa, pb, hs, dega, degb, b):
    return pl.pallas_call(
        _k_combine_stats_body,
        grid=(GRID,),
        in_specs=[
            pl.BlockSpec((BR, H), lambda i: (i, 0)),
            pl.BlockSpec((BR, H), lambda i: (i, 0)),
            pl.BlockSpec((BR, H), lambda i: (i, 0)),
            pl.BlockSpec((BR, 1), lambda i: (i, 0)),
            pl.BlockSpec((BR, 1), lambda i: (i, 0)),
            pl.BlockSpec((1, H), lambda i: (0, 0)),
        ],
        out_specs=[
            pl.BlockSpec((BR, H), lambda i: (i, 0)),
            pl.BlockSpec((8, H), lambda i: (0, 0)),
        ],
        out_shape=[
            jax.ShapeDtypeStruct((NPAD, H), jnp.float32),
            jax.ShapeDtypeStruct((8, H), jnp.float32),
        ],
    )(pa, pb, hs, dega, degb, b)


def _k_bn_mm_body(y_ref, st_ref, g_ref, be_ref, da_ref, db_ref, w_ref, o_ref):
    mu = st_ref[0:1, :] * (1.0 / N)
    ex2 = st_ref[1:2, :] * (1.0 / N)
    var = ex2 - mu * mu
    z = g_ref[...] * (y_ref[...] - mu) * lax.rsqrt(var + 1e-5) + be_ref[...]
    z = jnp.maximum(z, 0.0)
    dinv = _dinv_block(da_ref[...], db_ref[...])
    o_ref[...] = jnp.dot(z, w_ref[...], preferred_element_type=jnp.float32) * dinv


def _k_bn_mm(y, st, g, be, dega, degb, w):
    return pl.pallas_call(
        _k_bn_mm_body,
        grid=(GRID,),
        in_specs=[
            pl.BlockSpec((BR, H), lambda i: (i, 0)),
            pl.BlockSpec((8, H), lambda i: (0, 0)),
            pl.BlockSpec((1, H), lambda i: (0, 0)),
            pl.BlockSpec((1, H), lambda i: (0, 0)),
            pl.BlockSpec((BR, 1), lambda i: (i, 0)),
            pl.BlockSpec((BR, 1), lambda i: (i, 0)),
            pl.BlockSpec((H, H), lambda i: (0, 0)),
        ],
        out_specs=pl.BlockSpec((BR, H), lambda i: (i, 0)),
        out_shape=jax.ShapeDtypeStruct((NPAD, H), jnp.float32),
    )(y, st, g, be, dega, degb, w)


def _k_bn_proj_body(y_ref, st_ref, g_ref, be_ref, w_ref, o_ref):
    mu = st_ref[0:1, :] * (1.0 / N)
    ex2 = st_ref[1:2, :] * (1.0 / N)
    var = ex2 - mu * mu
    z = g_ref[...] * (y_ref[...] - mu) * lax.rsqrt(var + 1e-5) + be_ref[...]
    z = jnp.maximum(z, 0.0)
    o_ref[...] = jnp.dot(z, w_ref[...], preferred_element_type=jnp.float32)


def _k_bn_proj(y, st, g, be, wab):
    return pl.pallas_call(
        _k_bn_proj_body,
        grid=(GRID,),
        in_specs=[
            pl.BlockSpec((BR, H), lambda i: (i, 0)),
            pl.BlockSpec((8, H), lambda i: (0, 0)),
            pl.BlockSpec((1, H), lambda i: (0, 0)),
            pl.BlockSpec((1, H), lambda i: (0, 0)),
            pl.BlockSpec((H, H), lambda i: (0, 0)),
        ],
        out_specs=pl.BlockSpec((BR, H), lambda i: (i, 0)),
        out_shape=jax.ShapeDtypeStruct((NPAD, H), jnp.float32),
    )(y, st, g, be, wab)


def _k_edge_term_body(ea_ref, g_ref, bc_ref, o_ref):
    o_ref[...] = (jnp.dot(ea_ref[...], g_ref[...],
                          preferred_element_type=jnp.float32) + bc_ref[...])


def _k_edge_term(ea2, gmat, bc):
    return pl.pallas_call(
        _k_edge_term_body,
        grid=(GRID4,),
        in_specs=[
            pl.BlockSpec((BR4, 128), lambda i: (i, 0)),
            pl.BlockSpec((128, 8), lambda i: (0, 0)),
            pl.BlockSpec((1, 1), lambda i: (0, 0)),
        ],
        out_specs=pl.BlockSpec((BR4, 8), lambda i: (i, 0)),
        out_shape=jax.ShapeDtypeStruct((EB, 8), jnp.float32),
    )(ea2, gmat, bc)


def kernel(x, edge_index, edge_attr, W1, b1, g1, be1, W2, b2, g2, be2, Wc, bc):
    src = edge_index[0]
    dst = edge_index[1]

    xpad = jnp.pad(x, ((0, NPAD - N), (0, 0)))
    ones = jnp.ones((NPAD,), jnp.float32)

    degp = _sc_degree(dst, ones)
    dega = degp[0].reshape(NPAD, 1)
    degb = degp[1].reshape(NPAD, 1)

    b1r = b1.reshape(1, H)
    g1r = g1.reshape(1, H)
    be1r = be1.reshape(1, H)
    b2r = b2.reshape(1, H)
    g2r = g2.reshape(1, H)
    be2r = be2.reshape(1, H)

    hs1 = _k_mm_scale(xpad, W1, dega, degb)
    p1 = _sc_spmm(hs1, src, dst)
    y1, st1 = _k_combine_stats(p1[0], p1[1], hs1, dega, degb, b1r)
    hs2 = _k_bn_mm(y1, st1, g1r, be1r, dega, degb, W2)

    p2 = _sc_spmm(hs2, src, dst)
    y2, st2 = _k_combine_stats(p2[0], p2[1], hs2, dega, degb, b2r)

    wab = jnp.zeros((H, H), jnp.float32)
    wab = wab.at[:, 0].set(Wc[:H, 0])
    wab = wab.at[:, 1].set(Wc[H:2 * H, 0])
    ab = _k_bn_proj(y2, st2, g2r, be2r, wab)
    a = ab[:, 0]
    b = ab[:, 1]

    we = Wc[2 * H:, 0]
    gmat = jnp.zeros((128, 8), jnp.float32)
    for j in range(8):
        gmat = gmat.at[16 * j:16 * (j + 1), j].set(we)
    ea2 = edge_attr.reshape(EB, 128)
    et = _k_edge_term(ea2, gmat, bc.reshape(1, 1)).reshape(E)

    return _sc_edge_logits(a, b, src, dst, et)


# speedup vs baseline: 12.6019x; 12.6019x over previous
import functools

import jax
import jax.numpy as jnp
from jax import lax
from jax.experimental import pallas as pl
from jax.experimental.pallas import tpu as pltpu
from jax.experimental.pallas import tpu_sc as plsc

N = 10000
E = 320000
D = 128
DE = 16
H = 128

NC = 2
NS = 16
NW = NC * NS
L = 16

NPAD = 10240
RPT = NPAD // NS
C = 128
NCHUNK = E // C
CPT = -(-NCHUNK // NW)

BR = 256
GRID = NPAD // BR

EB = E * DE // 128
BR4 = 800
GRID4 = EB // BR4


def _sc_degree(dst, ones):
    mesh = plsc.VectorSubcoreMesh(core_axis_name="c", subcore_axis_name="s")

    @functools.partial(
        pl.kernel,
        out_type=jax.ShapeDtypeStruct((NC, NPAD), jnp.float32),
        mesh=mesh,
        scratch_types=[
            pltpu.VMEM_SHARED((NPAD,), jnp.float32),
            pltpu.VMEM((C,), jnp.int32),
            pltpu.VMEM((C,), jnp.float32),
        ],
    )
    def k(dst_hbm, ones_hbm, out_hbm, dsh, didx, ones_v):
        c = lax.axis_index("c")
        s = lax.axis_index("s")
        wid = s * NC + c
        rbase = s * RPT
        pltpu.sync_copy(ones_hbm.at[pl.ds(rbase, RPT)], dsh.at[pl.ds(rbase, RPT)])
        pltpu.sync_copy(ones_hbm.at[pl.ds(0, C)], ones_v)
        plsc.subcore_barrier()

        def body(j, carry):
            cid = j * NW + wid

            @pl.when(cid < NCHUNK)
            def _():
                pltpu.sync_copy(dst_hbm.at[pl.ds(cid * C, C)], didx)
                pltpu.sync_copy(ones_v, dsh.at[didx], add=True)

            return carry

        lax.fori_loop(0, CPT, body, 0)
        plsc.subcore_barrier()
        pltpu.sync_copy(dsh.at[pl.ds(rbase, RPT)],
                        out_hbm.at[c].at[pl.ds(rbase, RPT)])

    return k(dst, ones)


def _sc_spmm(hs, src, dst):
    mesh = plsc.VectorSubcoreMesh(core_axis_name="c", subcore_axis_name="s")

    @functools.partial(
        pl.kernel,
        out_type=jax.ShapeDtypeStruct((NC, NPAD, D), jnp.float32),
        mesh=mesh,
        scratch_types=[
            pltpu.VMEM_SHARED((NPAD, D), jnp.float32),
            pltpu.VMEM((C,), jnp.int32),
            pltpu.VMEM((C,), jnp.int32),
            pltpu.VMEM((C, D), jnp.float32),
            pltpu.SemaphoreType.DMA,
        ],
    )
    def k(hs_hbm, src_hbm, dst_hbm, out_hbm, ysh, sidx, didx, rows, sem):
        c = lax.axis_index("c")
        s = lax.axis_index("s")
        wid = s * NC + c
        rbase = s * RPT
        pltpu.sync_copy(hs_hbm.at[pl.ds(rbase, RPT)], ysh.at[pl.ds(rbase, RPT)])
        plsc.subcore_barrier()

        def body(j, carry):
            cid = j * NW + wid

            @pl.when(cid < NCHUNK)
            def _():
                start = cid * C
                pltpu.sync_copy(src_hbm.at[pl.ds(start, C)], sidx)
                pltpu.sync_copy(dst_hbm.at[pl.ds(start, C)], didx)
                pltpu.async_copy(hs_hbm.at[sidx], rows, sem).wait()
                pltpu.sync_copy(rows, ysh.at[didx], add=True)

            return carry

        lax.fori_loop(0, CPT, body, 0)
        plsc.subcore_barrier()
        pltpu.sync_copy(ysh.at[pl.ds(rbase, RPT)],
                        out_hbm.at[c].at[pl.ds(rbase, RPT)])

    return k(hs, src, dst)


def _sc_edge_logits(a, b, src, dst, et):
    mesh = plsc.VectorSubcoreMesh(core_axis_name="c", subcore_axis_name="s")

    @functools.partial(
        pl.kernel,
        out_type=jax.ShapeDtypeStruct((E,), jnp.float32),
        mesh=mesh,
        scratch_types=[
            pltpu.VMEM((NPAD,), jnp.float32),
            pltpu.VMEM((NPAD,), jnp.float32),
            pltpu.VMEM((C,), jnp.int32),
            pltpu.VMEM((C,), jnp.int32),
            pltpu.VMEM((C,), jnp.float32),
            pltpu.VMEM((C,), jnp.float32),
        ],
        compiler_params=pltpu.CompilerParams(needs_layout_passes=False),
    )
    def k(a_hbm, b_hbm, src_hbm, dst_hbm, et_hbm, out_hbm,
          av, bv, sidx, didx, etv, ov):
        c = lax.axis_index("c")
        s = lax.axis_index("s")
        wid = s * NC + c
        pltpu.sync_copy(a_hbm, av)
        pltpu.sync_copy(b_hbm, bv)

        def body(j, carry):
            cid = j * NW + wid

            @pl.when(cid < NCHUNK)
            def _():
                start = cid * C
                pltpu.sync_copy(src_hbm.at[pl.ds(start, C)], sidx)
                pltpu.sync_copy(dst_hbm.at[pl.ds(start, C)], didx)
                pltpu.sync_copy(et_hbm.at[pl.ds(start, C)], etv)
                for kk in range(C // L):
                    i16s = sidx[pl.ds(kk * L, L)]
                    i16d = didx[pl.ds(kk * L, L)]
                    ga = plsc.load_gather(av, [i16s])
                    gb = plsc.load_gather(bv, [i16d])
                    ov[pl.ds(kk * L, L)] = ga + gb + etv[pl.ds(kk * L, L)]
                pltpu.sync_copy(ov, out_hbm.at[pl.ds(start, C)])

            return carry

        lax.fori_loop(0, CPT, body, 0)

    return k(a, b, src, dst, et)


def _dinv_block(dega, degb):
    deg = dega + degb - 1.0
    return lax.rsqrt(jnp.maximum(deg, 1.0))


def _k_mm_scale_body(x_ref, w_ref, da_ref, db_ref, o_ref):
    dinv = _dinv_block(da_ref[...], db_ref[...])
    h = jnp.dot(x_ref[...], w_ref[...], preferred_element_type=jnp.float32)
    o_ref[...] = h * dinv


def _k_mm_scale(x, w, dega, degb):
    return pl.pallas_call(
        _k_mm_scale_body,
        grid=(GRID,),
        in_specs=[
            pl.BlockSpec((BR, D), lambda i: (i, 0)),
            pl.BlockSpec((D, H), lambda i: (0, 0)),
            pl.BlockSpec((BR, 1), lambda i: (i, 0)),
            pl.BlockSpec((BR, 1), lambda i: (i, 0)),
        ],
        out_specs=pl.BlockSpec((BR, H), lambda i: (i, 0)),
        out_shape=jax.ShapeDtypeStruct((NPAD, H), jnp.float32),
    )(x, w, dega, degb)


def _k_combine_stats_body(pa_ref, pb_ref, hs_ref, da_ref, db_ref, b_ref,
                          y_ref, st_ref):
    i = pl.program_id(0)
    dinv = _dinv_block(da_ref[...], db_ref[...])
    y = dinv * (pa_ref[...] + pb_ref[...] - hs_ref[...]) + b_ref[...]
    y_ref[...] = y

    @pl.when(i == 0)
    def _():
        st_ref[...] = jnp.zeros_like(st_ref)

    gid = i * BR + lax.broadcasted_iota(jnp.int32, (BR, 1), 0)
    ym = jnp.where(gid < N, y, 0.0)
    st_ref[0:1, :] += jnp.sum(ym, axis=0, keepdims=True)
    st_ref[1:2, :] += jnp.sum(ym * ym, axis=0, keepdims=True)


def _k_combine_stats(pa, pb, hs, dega, degb, b):
    return pl.pallas_call(
        _k_combine_stats_body,
        grid=(GRID,),
        in_specs=[
            pl.BlockSpec((BR, H), lambda i: (i, 0)),
            pl.BlockSpec((BR, H), lambda i: (i, 0)),
            pl.BlockSpec((BR, H), lambda i: (i, 0)),
            pl.BlockSpec((BR, 1), lambda i: (i, 0)),
            pl.BlockSpec((BR, 1), lambda i: (i, 0)),
            pl.BlockSpec((1, H), lambda i: (0, 0)),
        ],
        out_specs=[
            pl.BlockSpec((BR, H), lambda i: (i, 0)),
            pl.BlockSpec((8, H), lambda i: (0, 0)),
        ],
        out_shape=[
            jax.ShapeDtypeStruct((NPAD, H), jnp.float32),
            jax.ShapeDtypeStruct((8, H), jnp.float32),
        ],
    )(pa, pb, hs, dega, degb, b)


def _k_bn_mm_body(y_ref, st_ref, g_ref, be_ref, da_ref, db_ref, w_ref, o_ref):
    mu = st_ref[0:1, :] * (1.0 / N)
    ex2 = st_ref[1:2, :] * (1.0 / N)
    var = ex2 - mu * mu
    z = g_ref[...] * (y_ref[...] - mu) * lax.rsqrt(var + 1e-5) + be_ref[...]
    z = jnp.maximum(z, 0.0)
    dinv = _dinv_block(da_ref[...], db_ref[...])
    o_ref[...] = jnp.dot(z, w_ref[...], preferred_element_type=jnp.float32) * dinv


def _k_bn_mm(y, st, g, be, dega, degb, w):
    return pl.pallas_call(
        _k_bn_mm_body,
        grid=(GRID,),
        in_specs=[
            pl.BlockSpec((BR, H), lambda i: (i, 0)),
            pl.BlockSpec((8, H), lambda i: (0, 0)),
            pl.BlockSpec((1, H), lambda i: (0, 0)),
            pl.BlockSpec((1, H), lambda i: (0, 0)),
            pl.BlockSpec((BR, 1), lambda i: (i, 0)),
            pl.BlockSpec((BR, 1), lambda i: (i, 0)),
            pl.BlockSpec((H, H), lambda i: (0, 0)),
        ],
        out_specs=pl.BlockSpec((BR, H), lambda i: (i, 0)),
        out_shape=jax.ShapeDtypeStruct((NPAD, H), jnp.float32),
    )(y, st, g, be, dega, degb, w)


def _k_bn_proj_body(y_ref, st_ref, g_ref, be_ref, w_ref, o_ref):
    mu = st_ref[0:1, :] * (1.0 / N)
    ex2 = st_ref[1:2, :] * (1.0 / N)
    var = ex2 - mu * mu
    z = g_ref[...] * (y_ref[...] - mu) * lax.rsqrt(var + 1e-5) + be_ref[...]
    z = jnp.maximum(z, 0.0)
    o_ref[...] = jnp.dot(z, w_ref[...], preferred_element_type=jnp.float32)


def _k_bn_proj(y, st, g, be, wab):
    return pl.pallas_call(
        _k_bn_proj_body,
        grid=(GRID,),
        in_specs=[
            pl.BlockSpec((BR, H), lambda i: (i, 0)),
            pl.BlockSpec((8, H), lambda i: (0, 0)),
            pl.BlockSpec((1, H), lambda i: (0, 0)),
            pl.BlockSpec((1, H), lambda i: (0, 0)),
            pl.BlockSpec((H, H), lambda i: (0, 0)),
        ],
        out_specs=pl.BlockSpec((BR, H), lambda i: (i, 0)),
        out_shape=jax.ShapeDtypeStruct((NPAD, H), jnp.float32),
    )(y, st, g, be, wab)


def _k_edge_term_body(ea_ref, g_ref, bc_ref, o_ref):
    o_ref[...] = (jnp.dot(ea_ref[...], g_ref[...],
                          preferred_element_type=jnp.float32) + bc_ref[...])


def _k_edge_term(ea2, gmat, bc):
    return pl.pallas_call(
        _k_edge_term_body,
        grid=(GRID4,),
        in_specs=[
            pl.BlockSpec((BR4, 128), lambda i: (i, 0)),
            pl.BlockSpec((128, 8), lambda i: (0, 0)),
            pl.BlockSpec((1, 1), lambda i: (0, 0)),
        ],
        out_specs=pl.BlockSpec((BR4, 8), lambda i: (i, 0)),
        out_shape=jax.ShapeDtypeStruct((EB, 8), jnp.float32),
    )(ea2, gmat, bc)


def kernel(x, edge_index, edge_attr, W1, b1, g1, be1, W2, b2, g2, be2, Wc, bc):
    src = edge_index[0]
    dst = edge_index[1]

    xpad = jnp.pad(x, ((0, NPAD - N), (0, 0)))
    ones = jnp.ones((NPAD,), jnp.float32)

    degp = _sc_degree(dst, ones)
    dega = degp[0].reshape(NPAD, 1)
    degb = degp[1].reshape(NPAD, 1)

    b1r = b1.reshape(1, H)
    g1r = g1.reshape(1, H)
    be1r = be1.reshape(1, H)
    b2r = b2.reshape(1, H)
    g2r = g2.reshape(1, H)
    be2r = be2.reshape(1, H)

    hs1 = _k_mm_scale(xpad, W1, dega, degb)
    p1 = _sc_spmm(hs1, src, dst)
    y1, st1 = _k_combine_stats(p1[0], p1[1], hs1, dega, degb, b1r)
    hs2 = _k_bn_mm(y1, st1, g1r, be1r, dega, degb, W2)

    p2 = _sc_spmm(hs2, src, dst)
    y2, st2 = _k_combine_stats(p2[0], p2[1], hs2, dega, degb, b2r)

    wab = jnp.zeros((H, H), jnp.float32)
    wab = wab.at[:, 0].set(Wc[:H, 0])
    wab = wab.at[:, 1].set(Wc[H:2 * H, 0])
    ab = _k_bn_proj(y2, st2, g2r, be2r, wab)
    a = ab[:, 0]
    b = ab[:, 1]

    we = Wc[2 * H:, 0]
    gmat = jnp.zeros((128, 8), jnp.float32)
    for j in range(8):
        gmat = gmat.at[16 * j:16 * (j + 1), j].set(we)
    ea2 = edge_attr.reshape(EB, 128)
    et = _k_edge_term(ea2, gmat, bc.reshape(1, 1)).reshape(E)

    return _sc_edge_logits(a, b, src, dst, et)
